# Initial kernel scaffold; baseline (speedup 1.0000x reference)
import jax
import jax.numpy as jnp
from jax.experimental import pallas as pl


def kernel(support_embeddings, support_targets, query_embeddings, query_targets):
    raise NotImplementedError("write your pallas kernel here")


# speedup vs baseline: 3.1861x; 3.1861x over previous
import functools

import jax
import jax.numpy as jnp
from jax import lax
from jax.experimental import pallas as pl
from jax.experimental.pallas import tpu as pltpu
from jax.experimental.pallas import tpu_sc as plsc

N_CLASS = 64
D = 128
SUP_CHUNK = 80
Q_CHUNK = 40
NC = 2
NS = 16
NW = NC * NS


def _sc_stage(sup, st, qt, z128, z16, ones):
    n_sup = sup.shape[0]
    n_q = qt.shape[0]
    sup_per_w = n_sup // SUP_CHUNK // NW
    q_per_w = n_q // Q_CHUNK // NW
    mesh = plsc.VectorSubcoreMesh(core_axis_name="c", subcore_axis_name="s")

    @functools.partial(
        pl.kernel,
        mesh=mesh,
        out_type=[
            jax.ShapeDtypeStruct((NC, N_CLASS, D), jnp.float32),
            jax.ShapeDtypeStruct((NC, N_CLASS, 16), jnp.float32),
            jax.ShapeDtypeStruct((NC, N_CLASS, 16), jnp.float32),
        ],
        scratch_types=[
            pltpu.VMEM((SUP_CHUNK, D), jnp.float32),
            pltpu.VMEM((SUP_CHUNK,), jnp.int32),
            pltpu.VMEM((Q_CHUNK,), jnp.int32),
            pltpu.VMEM((SUP_CHUNK, 16), jnp.float32),
            pltpu.VMEM((Q_CHUNK, 16), jnp.float32),
            pltpu.VMEM_SHARED((N_CLASS, D), jnp.float32),
            pltpu.VMEM_SHARED((N_CLASS, 16), jnp.float32),
            pltpu.VMEM_SHARED((N_CLASS, 16), jnp.float32),
        ],
    )
    def k(sup_hbm, st_hbm, qt_hbm, z128_hbm, z16_hbm, ones_hbm,
          sums_out, cnt_out, qcnt_out,
          rows_v, sidx_v, qidx_v, ones_s, ones_q,
          acc_sh, cnt_sh, qcnt_sh):
        c = lax.axis_index("c")
        s = lax.axis_index("s")
        w = c * NS + s

        @pl.when(s == 0)
        def _init():
            pltpu.sync_copy(z128_hbm, acc_sh)
            pltpu.sync_copy(z16_hbm, cnt_sh)
            pltpu.sync_copy(z16_hbm, qcnt_sh)

        pltpu.sync_copy(ones_hbm, ones_s)
        pltpu.sync_copy(ones_hbm.at[pl.ds(0, Q_CHUNK)], ones_q)
        plsc.subcore_barrier()

        def sup_body(j, carry):
            base = (j * NW + w) * SUP_CHUNK
            pltpu.sync_copy(st_hbm.at[pl.ds(base, SUP_CHUNK)], sidx_v)
            pltpu.sync_copy(sup_hbm.at[pl.ds(base, SUP_CHUNK)], rows_v)
            pltpu.sync_copy(rows_v, acc_sh.at[sidx_v], add=True)
            pltpu.sync_copy(ones_s, cnt_sh.at[sidx_v], add=True)
            return carry

        lax.fori_loop(0, sup_per_w, sup_body, 0)

        def q_body(j, carry):
            base = (j * NW + w) * Q_CHUNK
            pltpu.sync_copy(qt_hbm.at[pl.ds(base, Q_CHUNK)], qidx_v)
            pltpu.sync_copy(ones_q, qcnt_sh.at[qidx_v], add=True)
            return carry

        lax.fori_loop(0, q_per_w, q_body, 0)

        plsc.subcore_barrier()

        @pl.when(s == 0)
        def _writeout():
            pltpu.sync_copy(acc_sh, sums_out.at[c])
            pltpu.sync_copy(cnt_sh, cnt_out.at[c])
            pltpu.sync_copy(qcnt_sh, qcnt_out.at[c])

    return k(sup, st, qt, z128, z16, ones)


def _tc_stage(q, sums2, cnt2, qcnt2):
    nq = q.shape[0]
    B = 2000
    grid = nq // B

    def body(q_ref, sums_ref, cnt_ref, qcnt_ref, out_ref):
        s = sums_ref[0] + sums_ref[1]
        cnt = cnt_ref[0, :, 0:1] + cnt_ref[1, :, 0:1]
        p = s / cnt
        ones_row = jnp.ones((1, D), jnp.float32)
        pn2 = lax.dot_general(ones_row, p * p, (((1,), (1,)), ((), ())),
                              preferred_element_type=jnp.float32)
        ones16 = jnp.ones((1, 16), jnp.float32)
        qc = qcnt_ref[0] + qcnt_ref[1]
        pres = lax.dot_general(ones16, qc, (((1,), (1,)), ((), ())),
                               preferred_element_type=jnp.float32)
        scale = jnp.where(pres > 0, 10.0, 0.0)
        qv = q_ref[...]
        qn2 = jnp.sum(qv * qv, axis=1, keepdims=True)
        dots = lax.dot_general(qv, p, (((1,), (1,)), ((), ())),
                               preferred_element_type=jnp.float32)
        denom = jnp.maximum(jnp.sqrt(qn2 * pn2), 1e-8)
        out_ref[...] = dots / denom * scale

    return pl.pallas_call(
        body,
        grid=(grid,),
        in_specs=[
            pl.BlockSpec((B, D), lambda i: (i, 0)),
            pl.BlockSpec((NC, N_CLASS, D), lambda i: (0, 0, 0)),
            pl.BlockSpec((NC, N_CLASS, 16), lambda i: (0, 0, 0)),
            pl.BlockSpec((NC, N_CLASS, 16), lambda i: (0, 0, 0)),
        ],
        out_specs=pl.BlockSpec((B, N_CLASS), lambda i: (i, 0)),
        out_shape=jax.ShapeDtypeStruct((nq, N_CLASS), jnp.float32),
    )(q, sums2, cnt2, qcnt2)


def kernel(support_embeddings, support_targets, query_embeddings, query_targets):
    z128 = jnp.zeros((N_CLASS, D), jnp.float32)
    z16 = jnp.zeros((N_CLASS, 16), jnp.float32)
    ones = jnp.ones((SUP_CHUNK, 16), jnp.float32)
    sums2, cnt2, qcnt2 = _sc_stage(
        support_embeddings, support_targets, query_targets, z128, z16, ones)
    return _tc_stage(query_embeddings, sums2, cnt2, qcnt2)


# speedup vs baseline: 3.7942x; 1.1909x over previous
import functools

import jax
import jax.numpy as jnp
from jax import lax
from jax.experimental import pallas as pl
from jax.experimental.pallas import tpu as pltpu
from jax.experimental.pallas import tpu_sc as plsc

N_CLASS = 64
D = 128
CH = 125
G = 2
ROWS_B = G * CH
NC = 2
NS = 16
NW = NC * NS


def _sc_stage(sup3, st3, qt3, z128, z16, ones, ident):
    nblk_s = sup3.shape[0] // NW
    nblk_q = qt3.shape[0] // NW
    mesh = plsc.VectorSubcoreMesh(core_axis_name="c", subcore_axis_name="s")

    @functools.partial(
        pl.kernel,
        mesh=mesh,
        out_type=[
            jax.ShapeDtypeStruct((NC, N_CLASS, D), jnp.float32),
            jax.ShapeDtypeStruct((NC, N_CLASS, 16), jnp.float32),
            jax.ShapeDtypeStruct((NC, N_CLASS, 16), jnp.float32),
        ],
        scratch_types=[
            pltpu.VMEM((2, ROWS_B, D), jnp.float32),
            pltpu.VMEM((2, G, CH), jnp.int32),
            pltpu.VMEM((2, G, CH), jnp.int32),
            pltpu.VMEM((CH, 16), jnp.float32),
            pltpu.VMEM((N_CLASS,), jnp.int32),
            pltpu.VMEM_SHARED((N_CLASS, D), jnp.float32),
            pltpu.VMEM_SHARED((N_CLASS, 16), jnp.float32),
            pltpu.VMEM_SHARED((N_CLASS, 16), jnp.float32),
            pltpu.SemaphoreType.DMA,
            pltpu.SemaphoreType.DMA,
            pltpu.SemaphoreType.DMA,
        ],
    )
    def k(sup_hbm, st_hbm, qt_hbm, z128_hbm, z16_hbm, ones_hbm, ident_hbm,
          sums_out, cnt_out, qcnt_out,
          rows_v, sidx_v, qidx_v, ones_s, ident_v,
          acc_sh, cnt_sh, qcnt_sh,
          ld0, ld1, sc_sem):
        c = lax.axis_index("c")
        s = lax.axis_index("s")
        w = c * NS + s
        lds = (ld0, ld1)

        @pl.when(s == 0)
        def _init_shared():
            pltpu.sync_copy(z128_hbm, acc_sh)
            pltpu.sync_copy(z16_hbm, cnt_sh)
            pltpu.sync_copy(z16_hbm, qcnt_sh)

        pltpu.sync_copy(ones_hbm, ones_s)
        pltpu.sync_copy(ident_hbm, ident_v)
        plsc.subcore_barrier()

        def s_issue(i, slot):
            g = i * NW + w
            pltpu.async_copy(st_hbm.at[g], sidx_v.at[slot], lds[slot])
            pltpu.async_copy(sup_hbm.at[g], rows_v.at[slot], lds[slot])

        def s_process(i, slot):
            pltpu.make_async_copy(st_hbm.at[0], sidx_v.at[slot],
                                  lds[slot]).wait()
            pltpu.make_async_copy(sup_hbm.at[0], rows_v.at[slot],
                                  lds[slot]).wait()
            cps = []
            for kk in range(G):
                cps.append(pltpu.async_copy(
                    rows_v.at[slot, pl.ds(kk * CH, CH)],
                    acc_sh.at[sidx_v.at[slot, kk]], sc_sem, add=True))
                cps.append(pltpu.async_copy(
                    ones_s, cnt_sh.at[sidx_v.at[slot, kk]], sc_sem, add=True))
            for cp in cps:
                cp.wait()

            @pl.when(i + 2 < nblk_s)
            def _refill():
                s_issue(i + 2, slot)

        s_issue(0, 0)
        s_issue(1, 1)

        def s_pair(t, carry):
            s_process(2 * t, 0)
            s_process(2 * t + 1, 1)
            return carry

        lax.fori_loop(0, nblk_s // 2, s_pair, 0)

        def q_issue(i, slot):
            g = i * NW + w
            pltpu.async_copy(qt_hbm.at[g], qidx_v.at[slot], lds[slot])

        def q_process(i, slot):
            pltpu.make_async_copy(qt_hbm.at[0], qidx_v.at[slot],
                                  lds[slot]).wait()
            cps = []
            for kk in range(G):
                cps.append(pltpu.async_copy(
                    ones_s, qcnt_sh.at[qidx_v.at[slot, kk]], sc_sem, add=True))
            for cp in cps:
                cp.wait()

            @pl.when(i + 2 < nblk_q)
            def _refill():
                q_issue(i + 2, slot)

        q_issue(0, 0)
        q_issue(1, 1)

        def q_pair(t, carry):
            q_process(2 * t, 0)
            q_process(2 * t + 1, 1)
            return carry

        lax.fori_loop(0, nblk_q // 2, q_pair, 0)

        plsc.subcore_barrier()

        @pl.when(s == 0)
        def _writeout():
            pltpu.sync_copy(acc_sh, sums_out.at[c])
            pltpu.sync_copy(cnt_sh, cnt_out.at[c])
            pltpu.sync_copy(qcnt_sh, qcnt_out.at[c])

    return k(sup3, st3, qt3, z128, z16, ones, ident)


def _tc_stage(q, sums2, cnt2, qcnt2):
    nq = q.shape[0]
    B = 2000
    grid = nq // B

    def body(q_ref, sums_ref, cnt_ref, qcnt_ref, out_ref):
        s = sums_ref[0] + sums_ref[1]
        cnt = cnt_ref[0, :, 0:1] + cnt_ref[1, :, 0:1]
        p = s / cnt
        ones_row = jnp.ones((1, D), jnp.float32)
        pn2 = lax.dot_general(ones_row, p * p, (((1,), (1,)), ((), ())),
                              preferred_element_type=jnp.float32)
        ones16 = jnp.ones((1, 16), jnp.float32)
        qc = qcnt_ref[0] + qcnt_ref[1]
        pres = lax.dot_general(ones16, qc, (((1,), (1,)), ((), ())),
                               preferred_element_type=jnp.float32)
        scale = jnp.where(pres > 0, 10.0, 0.0)
        qv = q_ref[...]
        qn2 = jnp.sum(qv * qv, axis=1, keepdims=True)
        dots = lax.dot_general(qv, p, (((1,), (1,)), ((), ())),
                               preferred_element_type=jnp.float32)
        denom = jnp.maximum(jnp.sqrt(qn2 * pn2), 1e-8)
        out_ref[...] = dots / denom * scale

    return pl.pallas_call(
        body,
        grid=(grid,),
        in_specs=[
            pl.BlockSpec((B, D), lambda i: (i, 0)),
            pl.BlockSpec((NC, N_CLASS, D), lambda i: (0, 0, 0)),
            pl.BlockSpec((NC, N_CLASS, 16), lambda i: (0, 0, 0)),
            pl.BlockSpec((NC, N_CLASS, 16), lambda i: (0, 0, 0)),
        ],
        out_specs=pl.BlockSpec((B, N_CLASS), lambda i: (i, 0)),
        out_shape=jax.ShapeDtypeStruct((nq, N_CLASS), jnp.float32),
    )(q, sums2, cnt2, qcnt2)


def kernel(support_embeddings, support_targets, query_embeddings, query_targets):
    z128 = jnp.zeros((N_CLASS, D), jnp.float32)
    z16 = jnp.zeros((N_CLASS, 16), jnp.float32)
    ones = jnp.ones((CH, 16), jnp.float32)
    ident = jnp.arange(N_CLASS, dtype=jnp.int32)
    sup3 = support_embeddings.reshape(-1, ROWS_B, D)
    st3 = support_targets.reshape(-1, G, CH)
    qt3 = query_targets.reshape(-1, G, CH)
    sums2, cnt2, qcnt2 = _sc_stage(sup3, st3, qt3, z128, z16, ones, ident)
    return _tc_stage(query_embeddings, sums2, cnt2, qcnt2)


# speedup vs baseline: 5.5324x; 1.4581x over previous
import functools

import jax
import jax.numpy as jnp
from jax import lax
from jax.experimental import pallas as pl
from jax.experimental.pallas import tpu as pltpu
from jax.experimental.pallas import tpu_sc as plsc

N_CLASS = 64
D = 128
CH_S = 100
G_S = 4
ROWS_B = G_S * CH_S
CH_Q = 125
G_Q = 2
QROWS_B = G_Q * CH_Q
NC = 2
NS = 16
NW = NC * NS


def _sc_stage(sup, st3, qt3, z128, z16, ones):
    nblk_s = st3.shape[0] // NW
    nblk_q = qt3.shape[0] // NW
    mesh = plsc.VectorSubcoreMesh(core_axis_name="c", subcore_axis_name="s")

    @functools.partial(
        pl.kernel,
        mesh=mesh,
        out_type=[
            jax.ShapeDtypeStruct((NC, N_CLASS, D), jnp.float32),
            jax.ShapeDtypeStruct((NC, N_CLASS, 16), jnp.float32),
            jax.ShapeDtypeStruct((NC, N_CLASS, 16), jnp.float32),
        ],
        scratch_types=[
            pltpu.VMEM((2, ROWS_B, D), jnp.float32),
            pltpu.VMEM((2, G_S, CH_S), jnp.int32),
            pltpu.VMEM((2, G_Q, CH_Q), jnp.int32),
            pltpu.VMEM((CH_Q, 16), jnp.float32),
            pltpu.VMEM_SHARED((N_CLASS, D), jnp.float32),
            pltpu.VMEM_SHARED((N_CLASS, 16), jnp.float32),
            pltpu.VMEM_SHARED((N_CLASS, 16), jnp.float32),
            pltpu.SemaphoreType.DMA,
            pltpu.SemaphoreType.DMA,
            pltpu.SemaphoreType.DMA,
        ],
    )
    def k(sup_hbm, st_hbm, qt_hbm, z128_hbm, z16_hbm, ones_hbm,
          sums_out, cnt_out, qcnt_out,
          rows_v, sidx_v, qidx_v, ones_s,
          acc_sh, cnt_sh, qcnt_sh,
          ld0, ld1, sc_sem):
        c = lax.axis_index("c")
        s = lax.axis_index("s")
        w = c * NS + s
        lds = (ld0, ld1)

        @pl.when(s == 0)
        def _init_shared():
            pltpu.sync_copy(z128_hbm, acc_sh)
            pltpu.sync_copy(z16_hbm, cnt_sh)
            pltpu.sync_copy(z16_hbm, qcnt_sh)

        pltpu.sync_copy(ones_hbm, ones_s)
        plsc.subcore_barrier()

        def s_issue(i, slot):
            g = i * NW + w
            pltpu.async_copy(st_hbm.at[g], sidx_v.at[slot], lds[slot])
            pltpu.async_copy(sup_hbm.at[pl.ds(g * ROWS_B, ROWS_B)],
                             rows_v.at[slot], lds[slot])

        def s_process(i, slot):
            pltpu.make_async_copy(st_hbm.at[0], sidx_v.at[slot],
                                  lds[slot]).wait()
            pltpu.make_async_copy(sup_hbm.at[pl.ds(0, ROWS_B)],
                                  rows_v.at[slot], lds[slot]).wait()
            cps = []
            for kk in range(G_S):
                cps.append(pltpu.async_copy(
                    rows_v.at[slot, pl.ds(kk * CH_S, CH_S)],
                    acc_sh.at[sidx_v.at[slot, kk]], sc_sem, add=True))
                cps.append(pltpu.async_copy(
                    ones_s.at[pl.ds(0, CH_S)],
                    cnt_sh.at[sidx_v.at[slot, kk]], sc_sem, add=True))
            for cp in cps:
                cp.wait()

            @pl.when(i + 2 < nblk_s)
            def _refill():
                s_issue(i + 2, slot)

        s_issue(0, 0)
        s_issue(1, 1)

        def s_pair(t, carry):
            s_process(2 * t, 0)
            s_process(2 * t + 1, 1)
            return carry

        lax.fori_loop(0, nblk_s // 2, s_pair, 0)
        if nblk_s % 2:
            s_process(nblk_s - 1, 0)

        def q_issue(i, slot):
            g = i * NW + w
            pltpu.async_copy(qt_hbm.at[g], qidx_v.at[slot], lds[slot])

        def q_process(i, slot):
            pltpu.make_async_copy(qt_hbm.at[0], qidx_v.at[slot],
                                  lds[slot]).wait()
            cps = []
            for kk in range(G_Q):
                cps.append(pltpu.async_copy(
                    ones_s, qcnt_sh.at[qidx_v.at[slot, kk]], sc_sem, add=True))
            for cp in cps:
                cp.wait()

            @pl.when(i + 2 < nblk_q)
            def _refill():
                q_issue(i + 2, slot)

        q_issue(0, 0)
        q_issue(1, 1)

        def q_pair(t, carry):
            q_process(2 * t, 0)
            q_process(2 * t + 1, 1)
            return carry

        lax.fori_loop(0, nblk_q // 2, q_pair, 0)

        plsc.subcore_barrier()

        @pl.when(s == 0)
        def _writeout():
            pltpu.sync_copy(acc_sh, sums_out.at[c])
            pltpu.sync_copy(cnt_sh, cnt_out.at[c])
            pltpu.sync_copy(qcnt_sh, qcnt_out.at[c])

    return k(sup, st3, qt3, z128, z16, ones)


def _tc_stage(q, sums2, cnt2, qcnt2):
    nq = q.shape[0]
    B = 2000
    grid = nq // B

    def body(q_ref, sums_ref, cnt_ref, qcnt_ref, out_ref):
        s = sums_ref[0] + sums_ref[1]
        cnt = cnt_ref[0, :, 0:1] + cnt_ref[1, :, 0:1]
        p = s / cnt
        ones_row = jnp.ones((1, D), jnp.float32)
        pn2 = lax.dot_general(ones_row, p * p, (((1,), (1,)), ((), ())),
                              preferred_element_type=jnp.float32)
        ones16 = jnp.ones((1, 16), jnp.float32)
        qc = qcnt_ref[0] + qcnt_ref[1]
        pres = lax.dot_general(ones16, qc, (((1,), (1,)), ((), ())),
                               preferred_element_type=jnp.float32)
        scale = jnp.where(pres > 0, 10.0, 0.0)
        qv = q_ref[...]
        qn2 = jnp.sum(qv * qv, axis=1, keepdims=True)
        dots = lax.dot_general(qv, p, (((1,), (1,)), ((), ())),
                               preferred_element_type=jnp.float32)
        denom = jnp.maximum(jnp.sqrt(qn2 * pn2), 1e-8)
        out_ref[...] = dots / denom * scale

    return pl.pallas_call(
        body,
        grid=(grid,),
        in_specs=[
            pl.BlockSpec((B, D), lambda i: (i, 0)),
            pl.BlockSpec((NC, N_CLASS, D), lambda i: (0, 0, 0)),
            pl.BlockSpec((NC, N_CLASS, 16), lambda i: (0, 0, 0)),
            pl.BlockSpec((NC, N_CLASS, 16), lambda i: (0, 0, 0)),
        ],
        out_specs=pl.BlockSpec((B, N_CLASS), lambda i: (i, 0)),
        out_shape=jax.ShapeDtypeStruct((nq, N_CLASS), jnp.float32),
    )(q, sums2, cnt2, qcnt2)


def kernel(support_embeddings, support_targets, query_embeddings, query_targets):
    z128 = jnp.zeros((N_CLASS, D), jnp.float32)
    z16 = jnp.zeros((N_CLASS, 16), jnp.float32)
    ones = jnp.ones((CH_Q, 16), jnp.float32)
    st3 = support_targets.reshape(-1, G_S, CH_S)
    qt3 = query_targets.reshape(-1, G_Q, CH_Q)
    sums2, cnt2, qcnt2 = _sc_stage(
        support_embeddings, st3, qt3, z128, z16, ones)
    return _tc_stage(query_embeddings, sums2, cnt2, qcnt2)


# speedup vs baseline: 6.1369x; 1.1092x over previous
import functools

import jax
import jax.numpy as jnp
from jax import lax
from jax.experimental import pallas as pl
from jax.experimental.pallas import tpu as pltpu
from jax.experimental.pallas import tpu_sc as plsc

N_CLASS = 64
D = 128
CH_S = 100
G_S = 4
ROWS_B = G_S * CH_S
NC = 2
NS = 16
NW = NC * NS

T_BLK = 2000
N_SUP_BLKS = 160
CUT = 96
N_QT_BLKS = 80


def _sc_stage(sup, st3, z128):
    nblk_s = st3.shape[0] // NW
    mesh = plsc.VectorSubcoreMesh(core_axis_name="c", subcore_axis_name="s")

    @functools.partial(
        pl.kernel,
        mesh=mesh,
        out_type=jax.ShapeDtypeStruct((NC, N_CLASS, D), jnp.float32),
        scratch_types=[
            pltpu.VMEM((2, ROWS_B, D), jnp.float32),
            pltpu.VMEM((2, G_S, CH_S), jnp.int32),
            pltpu.VMEM_SHARED((N_CLASS, D), jnp.float32),
            pltpu.SemaphoreType.DMA,
            pltpu.SemaphoreType.DMA,
            pltpu.SemaphoreType.DMA,
        ],
    )
    def k(sup_hbm, st_hbm, z128_hbm, sums_out,
          rows_v, sidx_v, acc_sh, ld0, ld1, sc_sem):
        c = lax.axis_index("c")
        s = lax.axis_index("s")
        w = c * NS + s
        lds = (ld0, ld1)

        @pl.when(s == 0)
        def _init_shared():
            pltpu.sync_copy(z128_hbm, acc_sh)

        plsc.subcore_barrier()

        def s_issue(i, slot):
            g = i * NW + w
            pltpu.async_copy(st_hbm.at[g], sidx_v.at[slot], lds[slot])
            pltpu.async_copy(sup_hbm.at[pl.ds(g * ROWS_B, ROWS_B)],
                             rows_v.at[slot], lds[slot])

        def s_process(i, slot):
            pltpu.make_async_copy(st_hbm.at[0], sidx_v.at[slot],
                                  lds[slot]).wait()
            pltpu.make_async_copy(sup_hbm.at[pl.ds(0, ROWS_B)],
                                  rows_v.at[slot], lds[slot]).wait()
            cps = []
            for kk in range(G_S):
                cps.append(pltpu.async_copy(
                    rows_v.at[slot, pl.ds(kk * CH_S, CH_S)],
                    acc_sh.at[sidx_v.at[slot, kk]], sc_sem, add=True))
            for cp in cps:
                cp.wait()

            @pl.when(i + 2 < nblk_s)
            def _refill():
                s_issue(i + 2, slot)

        s_issue(0, 0)
        s_issue(1, 1)

        def s_pair(t, carry):
            s_process(2 * t, 0)
            s_process(2 * t + 1, 1)
            return carry

        lax.fori_loop(0, nblk_s // 2, s_pair, 0)
        if nblk_s % 2:
            s_process(nblk_s - 1, 0)

        plsc.subcore_barrier()

        @pl.when(s == 0)
        def _writeout():
            pltpu.sync_copy(acc_sh, sums_out.at[c])

    return k(sup, st3, z128)


def _tc_pre(sup3, st3, qt3):
    def body(sup_ref, st_ref, qt_ref, psum_ref, cnt_ref, qcnt_ref):
        i = pl.program_id(0)

        @pl.when(i == 0)
        def _init():
            psum_ref[...] = jnp.zeros_like(psum_ref)
            cnt_ref[...] = jnp.zeros_like(cnt_ref)
            qcnt_ref[...] = jnp.zeros_like(qcnt_ref)

        iota = lax.broadcasted_iota(jnp.int32, (N_CLASS, T_BLK), 0)
        ones8 = jnp.ones((T_BLK, 8), jnp.float32)
        oh_st = (iota == st_ref[0]).astype(jnp.float32)
        cnt_ref[...] += lax.dot_general(
            oh_st, ones8, (((1,), (0,)), ((), ())),
            preferred_element_type=jnp.float32)

        @pl.when(i < N_QT_BLKS)
        def _qhist():
            oh_qt = (iota == qt_ref[0]).astype(jnp.float32)
            qcnt_ref[...] += lax.dot_general(
                oh_qt, ones8, (((1,), (0,)), ((), ())),
                preferred_element_type=jnp.float32)

        @pl.when(i >= CUT)
        def _psum():
            psum_ref[...] += lax.dot_general(
                oh_st, sup_ref[0], (((1,), (0,)), ((), ())),
                preferred_element_type=jnp.float32)

    return pl.pallas_call(
        body,
        grid=(N_SUP_BLKS,),
        in_specs=[
            pl.BlockSpec((1, T_BLK, D),
                         lambda i: (jnp.maximum(i, CUT), 0, 0)),
            pl.BlockSpec((1, 1, T_BLK), lambda i: (i, 0, 0)),
            pl.BlockSpec((1, 1, T_BLK),
                         lambda i: (jnp.minimum(i, N_QT_BLKS - 1), 0, 0)),
        ],
        out_specs=[
            pl.BlockSpec((N_CLASS, D), lambda i: (0, 0)),
            pl.BlockSpec((N_CLASS, 8), lambda i: (0, 0)),
            pl.BlockSpec((N_CLASS, 8), lambda i: (0, 0)),
        ],
        out_shape=[
            jax.ShapeDtypeStruct((N_CLASS, D), jnp.float32),
            jax.ShapeDtypeStruct((N_CLASS, 8), jnp.float32),
            jax.ShapeDtypeStruct((N_CLASS, 8), jnp.float32),
        ],
    )(sup3, st3, qt3)


def _tc_final(q, sums2, psum, cnt, qcnt):
    nq = q.shape[0]
    B = 16000
    grid = nq // B

    def body(q_ref, sums_ref, psum_ref, cnt_ref, qcnt_ref, out_ref):
        s = sums_ref[0] + sums_ref[1] + psum_ref[...]
        cntc = cnt_ref[:, 0:1]
        p = s / cntc
        ones_row = jnp.ones((1, D), jnp.float32)
        pn2 = lax.dot_general(ones_row, p * p, (((1,), (1,)), ((), ())),
                              preferred_element_type=jnp.float32)
        ones8 = jnp.ones((1, 8), jnp.float32)
        pres = lax.dot_general(ones8, qcnt_ref[...], (((1,), (1,)), ((), ())),
                               preferred_element_type=jnp.float32)
        scale = jnp.where(pres > 0, 10.0, 0.0)
        qv = q_ref[...]
        qn2 = jnp.sum(qv * qv, axis=1, keepdims=True)
        dots = lax.dot_general(qv, p, (((1,), (1,)), ((), ())),
                               preferred_element_type=jnp.float32)
        denom = jnp.maximum(jnp.sqrt(qn2 * pn2), 1e-8)
        out_ref[...] = dots / denom * scale

    return pl.pallas_call(
        body,
        grid=(grid,),
        in_specs=[
            pl.BlockSpec((B, D), lambda i: (i, 0)),
            pl.BlockSpec((NC, N_CLASS, D), lambda i: (0, 0, 0)),
            pl.BlockSpec((N_CLASS, D), lambda i: (0, 0)),
            pl.BlockSpec((N_CLASS, 8), lambda i: (0, 0)),
            pl.BlockSpec((N_CLASS, 8), lambda i: (0, 0)),
        ],
        out_specs=pl.BlockSpec((B, N_CLASS), lambda i: (i, 0)),
        out_shape=jax.ShapeDtypeStruct((nq, N_CLASS), jnp.float32),
    )(q, sums2, psum, cnt, qcnt)


def kernel(support_embeddings, support_targets, query_embeddings, query_targets):
    n_sc = CUT * T_BLK
    z128 = jnp.zeros((N_CLASS, D), jnp.float32)
    st3_sc = support_targets[:n_sc].reshape(-1, G_S, CH_S)
    sup3 = support_embeddings.reshape(N_SUP_BLKS, T_BLK, D)
    st3 = support_targets.reshape(N_SUP_BLKS, 1, T_BLK)
    qt3 = query_targets.reshape(N_QT_BLKS, 1, T_BLK)
    sums2 = _sc_stage(support_embeddings, st3_sc, z128)
    psum, cnt, qcnt = _tc_pre(sup3, st3, qt3)
    return _tc_final(query_embeddings, sums2, psum, cnt, qcnt)


# speedup vs baseline: 6.3754x; 1.0389x over previous
import functools

import jax
import jax.numpy as jnp
from jax import lax
from jax.experimental import pallas as pl
from jax.experimental.pallas import tpu as pltpu
from jax.experimental.pallas import tpu_sc as plsc

N_CLASS = 64
D = 128
CH_S = 100
G_S = 4
ROWS_B = G_S * CH_S
NC = 2
NS = 16
NW = NC * NS

T_BLK = 2000
N_SUP_BLKS = 160
CUT = 128
N_QT_BLKS = 80


def _sc_stage(sup, st3, z128):
    nblk_s = st3.shape[0] // NW
    mesh = plsc.VectorSubcoreMesh(core_axis_name="c", subcore_axis_name="s")

    @functools.partial(
        pl.kernel,
        mesh=mesh,
        out_type=jax.ShapeDtypeStruct((NC, N_CLASS, D), jnp.float32),
        scratch_types=[
            pltpu.VMEM((2, ROWS_B, D), jnp.float32),
            pltpu.VMEM((2, G_S, CH_S), jnp.int32),
            pltpu.VMEM_SHARED((N_CLASS, D), jnp.float32),
            pltpu.SemaphoreType.DMA,
            pltpu.SemaphoreType.DMA,
            pltpu.SemaphoreType.DMA,
        ],
    )
    def k(sup_hbm, st_hbm, z128_hbm, sums_out,
          rows_v, sidx_v, acc_sh, ld0, ld1, sc_sem):
        c = lax.axis_index("c")
        s = lax.axis_index("s")
        w = c * NS + s
        lds = (ld0, ld1)

        @pl.when(s == 0)
        def _init_shared():
            pltpu.sync_copy(z128_hbm, acc_sh)

        plsc.subcore_barrier()

        def s_issue(i, slot):
            g = i * NW + w
            pltpu.async_copy(st_hbm.at[g], sidx_v.at[slot], lds[slot])
            pltpu.async_copy(sup_hbm.at[pl.ds(g * ROWS_B, ROWS_B)],
                             rows_v.at[slot], lds[slot])

        def s_process(i, slot):
            pltpu.make_async_copy(st_hbm.at[0], sidx_v.at[slot],
                                  lds[slot]).wait()
            pltpu.make_async_copy(sup_hbm.at[pl.ds(0, ROWS_B)],
                                  rows_v.at[slot], lds[slot]).wait()
            cps = []
            for kk in range(G_S):
                cps.append(pltpu.async_copy(
                    rows_v.at[slot, pl.ds(kk * CH_S, CH_S)],
                    acc_sh.at[sidx_v.at[slot, kk]], sc_sem, add=True))
            for cp in cps:
                cp.wait()

            @pl.when(i + 2 < nblk_s)
            def _refill():
                s_issue(i + 2, slot)

        s_issue(0, 0)
        s_issue(1, 1)

        def s_pair(t, carry):
            s_process(2 * t, 0)
            s_process(2 * t + 1, 1)
            return carry

        lax.fori_loop(0, nblk_s // 2, s_pair, 0)
        if nblk_s % 2:
            s_process(nblk_s - 1, 0)

        plsc.subcore_barrier()

        @pl.when(s == 0)
        def _writeout():
            pltpu.sync_copy(acc_sh, sums_out.at[c])

    return k(sup, st3, z128)


def _tc_pre(sup3, st3, qt3):
    def body(sup_ref, st_ref, qt_ref, psum_ref, cnt_ref, qcnt_ref):
        i = pl.program_id(0)

        @pl.when(i == 0)
        def _init():
            psum_ref[...] = jnp.zeros_like(psum_ref)
            cnt_ref[...] = jnp.zeros_like(cnt_ref)
            qcnt_ref[...] = jnp.zeros_like(qcnt_ref)

        iota = lax.broadcasted_iota(jnp.int32, (N_CLASS, T_BLK), 0)
        ones8 = jnp.ones((T_BLK, 8), jnp.float32)
        oh_st = (iota == st_ref[0]).astype(jnp.float32)
        cnt_ref[...] += lax.dot_general(
            oh_st, ones8, (((1,), (0,)), ((), ())),
            preferred_element_type=jnp.float32)

        @pl.when(i < N_QT_BLKS)
        def _qhist():
            oh_qt = (iota == qt_ref[0]).astype(jnp.float32)
            qcnt_ref[...] += lax.dot_general(
                oh_qt, ones8, (((1,), (0,)), ((), ())),
                preferred_element_type=jnp.float32)

        @pl.when(i >= CUT)
        def _psum():
            psum_ref[...] += lax.dot_general(
                oh_st, sup_ref[0], (((1,), (0,)), ((), ())),
                preferred_element_type=jnp.float32)

    return pl.pallas_call(
        body,
        grid=(N_SUP_BLKS,),
        in_specs=[
            pl.BlockSpec((1, T_BLK, D),
                         lambda i: (jnp.maximum(i, CUT), 0, 0)),
            pl.BlockSpec((1, 1, T_BLK), lambda i: (i, 0, 0)),
            pl.BlockSpec((1, 1, T_BLK),
                         lambda i: (jnp.minimum(i, N_QT_BLKS - 1), 0, 0)),
        ],
        out_specs=[
            pl.BlockSpec((N_CLASS, D), lambda i: (0, 0)),
            pl.BlockSpec((N_CLASS, 8), lambda i: (0, 0)),
            pl.BlockSpec((N_CLASS, 8), lambda i: (0, 0)),
        ],
        out_shape=[
            jax.ShapeDtypeStruct((N_CLASS, D), jnp.float32),
            jax.ShapeDtypeStruct((N_CLASS, 8), jnp.float32),
            jax.ShapeDtypeStruct((N_CLASS, 8), jnp.float32),
        ],
    )(sup3, st3, qt3)


def _tc_final(q, sums2, psum, cnt, qcnt):
    nq = q.shape[0]
    B = 16000
    grid = nq // B

    def body(q_ref, sums_ref, psum_ref, cnt_ref, qcnt_ref, out_ref):
        s = sums_ref[0] + sums_ref[1] + psum_ref[...]
        cntc = cnt_ref[:, 0:1]
        p = s / cntc
        ones_row = jnp.ones((1, D), jnp.float32)
        pn2 = lax.dot_general(ones_row, p * p, (((1,), (1,)), ((), ())),
                              preferred_element_type=jnp.float32)
        ones8 = jnp.ones((1, 8), jnp.float32)
        pres = lax.dot_general(ones8, qcnt_ref[...], (((1,), (1,)), ((), ())),
                               preferred_element_type=jnp.float32)
        scale = jnp.where(pres > 0, 10.0, 0.0)
        qv = q_ref[...]
        qn2 = jnp.sum(qv * qv, axis=1, keepdims=True)
        dots = lax.dot_general(qv, p, (((1,), (1,)), ((), ())),
                               preferred_element_type=jnp.float32)
        denom = jnp.maximum(jnp.sqrt(qn2 * pn2), 1e-8)
        out_ref[...] = dots / denom * scale

    return pl.pallas_call(
        body,
        grid=(grid,),
        in_specs=[
            pl.BlockSpec((B, D), lambda i: (i, 0)),
            pl.BlockSpec((NC, N_CLASS, D), lambda i: (0, 0, 0)),
            pl.BlockSpec((N_CLASS, D), lambda i: (0, 0)),
            pl.BlockSpec((N_CLASS, 8), lambda i: (0, 0)),
            pl.BlockSpec((N_CLASS, 8), lambda i: (0, 0)),
        ],
        out_specs=pl.BlockSpec((B, N_CLASS), lambda i: (i, 0)),
        out_shape=jax.ShapeDtypeStruct((nq, N_CLASS), jnp.float32),
    )(q, sums2, psum, cnt, qcnt)


def kernel(support_embeddings, support_targets, query_embeddings, query_targets):
    n_sc = CUT * T_BLK
    z128 = jnp.zeros((N_CLASS, D), jnp.float32)
    st3_sc = support_targets[:n_sc].reshape(-1, G_S, CH_S)
    sup3 = support_embeddings.reshape(N_SUP_BLKS, T_BLK, D)
    st3 = support_targets.reshape(N_SUP_BLKS, 1, T_BLK)
    qt3 = query_targets.reshape(N_QT_BLKS, 1, T_BLK)
    sums2 = _sc_stage(support_embeddings, st3_sc, z128)
    psum, cnt, qcnt = _tc_pre(sup3, st3, qt3)
    return _tc_final(query_embeddings, sums2, psum, cnt, qcnt)


# speedup vs baseline: 7.5075x; 1.1776x over previous
import functools

import jax
import jax.numpy as jnp
from jax import lax
from jax.experimental import pallas as pl
from jax.experimental.pallas import tpu as pltpu
from jax.experimental.pallas import tpu_sc as plsc

N_CLASS = 64
D = 128
CH_S = 100
G_S = 4
ROWS_B = G_S * CH_S
NC = 2
NS = 16
NW = NC * NS

TB2 = 2000
RB2 = 8 * TB2
N_SUP_BLKS = 20
CUT_B = 16
N_QT_BLKS = 10


def _sc_stage(sup, st3, z128):
    nblk_s = st3.shape[0] // NW
    mesh = plsc.VectorSubcoreMesh(core_axis_name="c", subcore_axis_name="s")

    @functools.partial(
        pl.kernel,
        mesh=mesh,
        out_type=jax.ShapeDtypeStruct((NC, N_CLASS, D), jnp.float32),
        scratch_types=[
            pltpu.VMEM((2, ROWS_B, D), jnp.float32),
            pltpu.VMEM((2, G_S, CH_S), jnp.int32),
            pltpu.VMEM_SHARED((N_CLASS, D), jnp.float32),
            pltpu.SemaphoreType.DMA,
            pltpu.SemaphoreType.DMA,
            pltpu.SemaphoreType.DMA,
        ],
    )
    def k(sup_hbm, st_hbm, z128_hbm, sums_out,
          rows_v, sidx_v, acc_sh, ld0, ld1, sc_sem):
        c = lax.axis_index("c")
        s = lax.axis_index("s")
        w = c * NS + s
        lds = (ld0, ld1)

        @pl.when(s == 0)
        def _init_shared():
            pltpu.sync_copy(z128_hbm, acc_sh)

        plsc.subcore_barrier()

        def s_issue(i, slot):
            g = i * NW + w
            pltpu.async_copy(st_hbm.at[g], sidx_v.at[slot], lds[slot])
            pltpu.async_copy(sup_hbm.at[pl.ds(g * ROWS_B, ROWS_B)],
                             rows_v.at[slot], lds[slot])

        def s_process(i, slot):
            pltpu.make_async_copy(st_hbm.at[0], sidx_v.at[slot],
                                  lds[slot]).wait()
            pltpu.make_async_copy(sup_hbm.at[pl.ds(0, ROWS_B)],
                                  rows_v.at[slot], lds[slot]).wait()
            cps = []
            for kk in range(G_S):
                cps.append(pltpu.async_copy(
                    rows_v.at[slot, pl.ds(kk * CH_S, CH_S)],
                    acc_sh.at[sidx_v.at[slot, kk]], sc_sem, add=True))
            for cp in cps:
                cp.wait()

            @pl.when(i + 2 < nblk_s)
            def _refill():
                s_issue(i + 2, slot)

        s_issue(0, 0)
        s_issue(1, 1)

        def s_pair(t, carry):
            s_process(2 * t, 0)
            s_process(2 * t + 1, 1)
            return carry

        lax.fori_loop(0, nblk_s // 2, s_pair, 0)
        if nblk_s % 2:
            s_process(nblk_s - 1, 0)

        plsc.subcore_barrier()

        @pl.when(s == 0)
        def _writeout():
            pltpu.sync_copy(acc_sh, sums_out.at[c])

    return k(sup, st3, z128)


def _tc_pre(sup3, st3, qt3):
    def body(sup_ref, st_ref, qt_ref, psum_ref, cnt_ref, qcnt_ref):
        i = pl.program_id(0)

        @pl.when(i == 0)
        def _init():
            psum_ref[...] = jnp.zeros_like(psum_ref)
            cnt_ref[...] = jnp.zeros_like(cnt_ref)
            qcnt_ref[...] = jnp.zeros_like(qcnt_ref)

        iota = lax.broadcasted_iota(jnp.int32, (N_CLASS, TB2), 0)
        ones8 = jnp.ones((TB2, 8), jnp.float32)
        t8 = st_ref[0]
        ohsum = None
        for r in range(8):
            oh_r = (iota == t8[r:r + 1, :]).astype(jnp.float32)
            ohsum = oh_r if r == 0 else ohsum + oh_r

            @pl.when(i >= CUT_B)
            def _psum(oh_r=oh_r, r=r):
                psum_ref[...] += lax.dot_general(
                    oh_r, sup_ref[0, pl.ds(r * TB2, TB2)],
                    (((1,), (0,)), ((), ())),
                    preferred_element_type=jnp.float32)

        cnt_ref[...] += lax.dot_general(
            ohsum, ones8, (((1,), (0,)), ((), ())),
            preferred_element_type=jnp.float32)

        @pl.when(i < N_QT_BLKS)
        def _qhist():
            q8 = qt_ref[0]
            qsum = None
            for r in range(8):
                oh_r = (iota == q8[r:r + 1, :]).astype(jnp.float32)
                qsum = oh_r if r == 0 else qsum + oh_r
            qcnt_ref[...] += lax.dot_general(
                qsum, ones8, (((1,), (0,)), ((), ())),
                preferred_element_type=jnp.float32)

    return pl.pallas_call(
        body,
        grid=(N_SUP_BLKS,),
        in_specs=[
            pl.BlockSpec((1, RB2, D),
                         lambda i: (jnp.maximum(i, CUT_B), 0, 0)),
            pl.BlockSpec((1, 8, TB2), lambda i: (i, 0, 0)),
            pl.BlockSpec((1, 8, TB2),
                         lambda i: (jnp.minimum(i, N_QT_BLKS - 1), 0, 0)),
        ],
        out_specs=[
            pl.BlockSpec((N_CLASS, D), lambda i: (0, 0)),
            pl.BlockSpec((N_CLASS, 8), lambda i: (0, 0)),
            pl.BlockSpec((N_CLASS, 8), lambda i: (0, 0)),
        ],
        out_shape=[
            jax.ShapeDtypeStruct((N_CLASS, D), jnp.float32),
            jax.ShapeDtypeStruct((N_CLASS, 8), jnp.float32),
            jax.ShapeDtypeStruct((N_CLASS, 8), jnp.float32),
        ],
    )(sup3, st3, qt3)


def _tc_final(q, sums2, psum, cnt, qcnt):
    nq = q.shape[0]
    B = 16000
    grid = nq // B

    def body(q_ref, sums_ref, psum_ref, cnt_ref, qcnt_ref, out_ref):
        s = sums_ref[0] + sums_ref[1] + psum_ref[...]
        cntc = cnt_ref[:, 0:1]
        p = s / cntc
        ones_row = jnp.ones((1, D), jnp.float32)
        pn2 = lax.dot_general(ones_row, p * p, (((1,), (1,)), ((), ())),
                              preferred_element_type=jnp.float32)
        ones8 = jnp.ones((1, 8), jnp.float32)
        pres = lax.dot_general(ones8, qcnt_ref[...], (((1,), (1,)), ((), ())),
                               preferred_element_type=jnp.float32)
        scale = jnp.where(pres > 0, 10.0, 0.0)
        qv = q_ref[...]
        qn2 = jnp.sum(qv * qv, axis=1, keepdims=True)
        dots = lax.dot_general(qv, p, (((1,), (1,)), ((), ())),
                               preferred_element_type=jnp.float32)
        denom = jnp.maximum(jnp.sqrt(qn2 * pn2), 1e-8)
        out_ref[...] = dots / denom * scale

    return pl.pallas_call(
        body,
        grid=(grid,),
        in_specs=[
            pl.BlockSpec((B, D), lambda i: (i, 0)),
            pl.BlockSpec((NC, N_CLASS, D), lambda i: (0, 0, 0)),
            pl.BlockSpec((N_CLASS, D), lambda i: (0, 0)),
            pl.BlockSpec((N_CLASS, 8), lambda i: (0, 0)),
            pl.BlockSpec((N_CLASS, 8), lambda i: (0, 0)),
        ],
        out_specs=pl.BlockSpec((B, N_CLASS), lambda i: (i, 0)),
        out_shape=jax.ShapeDtypeStruct((nq, N_CLASS), jnp.float32),
    )(q, sums2, psum, cnt, qcnt)


def kernel(support_embeddings, support_targets, query_embeddings, query_targets):
    n_sc = CUT_B * RB2
    z128 = jnp.zeros((N_CLASS, D), jnp.float32)
    st3_sc = support_targets[:n_sc].reshape(-1, G_S, CH_S)
    sup3 = support_embeddings.reshape(N_SUP_BLKS, RB2, D)
    st3 = support_targets.reshape(N_SUP_BLKS, 8, TB2)
    qt3 = query_targets.reshape(N_QT_BLKS, 8, TB2)
    sums2 = _sc_stage(support_embeddings, st3_sc, z128)
    psum, cnt, qcnt = _tc_pre(sup3, st3, qt3)
    return _tc_final(query_embeddings, sums2, psum, cnt, qcnt)


# speedup vs baseline: 8.0521x; 1.0725x over previous
import functools

import jax
import jax.numpy as jnp
from jax import lax
from jax.experimental import pallas as pl
from jax.experimental.pallas import tpu as pltpu
from jax.experimental.pallas import tpu_sc as plsc

N_CLASS = 64
D = 128
CH_S = 100
G_S = 4
ROWS_B = G_S * CH_S
NC = 2
NS = 16
NW = NC * NS

TB2 = 2000
RB2 = 8 * TB2
N_SUP_BLKS = 20
CUT_B = 12
N_QT_BLKS = 10


def _sc_stage(sup, st3, z128):
    nblk_s = st3.shape[0] // NW
    mesh = plsc.VectorSubcoreMesh(core_axis_name="c", subcore_axis_name="s")

    @functools.partial(
        pl.kernel,
        mesh=mesh,
        out_type=jax.ShapeDtypeStruct((NC, N_CLASS, D), jnp.float32),
        scratch_types=[
            pltpu.VMEM((2, ROWS_B, D), jnp.float32),
            pltpu.VMEM((2, G_S, CH_S), jnp.int32),
            pltpu.VMEM_SHARED((N_CLASS, D), jnp.float32),
            pltpu.SemaphoreType.DMA,
            pltpu.SemaphoreType.DMA,
            pltpu.SemaphoreType.DMA,
        ],
    )
    def k(sup_hbm, st_hbm, z128_hbm, sums_out,
          rows_v, sidx_v, acc_sh, ld0, ld1, sc_sem):
        c = lax.axis_index("c")
        s = lax.axis_index("s")
        w = c * NS + s
        lds = (ld0, ld1)

        @pl.when(s == 0)
        def _init_shared():
            pltpu.sync_copy(z128_hbm, acc_sh)

        plsc.subcore_barrier()

        def s_issue(i, slot):
            g = i * NW + w
            pltpu.async_copy(st_hbm.at[g], sidx_v.at[slot], lds[slot])
            pltpu.async_copy(sup_hbm.at[pl.ds(g * ROWS_B, ROWS_B)],
                             rows_v.at[slot], lds[slot])

        def s_process(i, slot):
            pltpu.make_async_copy(st_hbm.at[0], sidx_v.at[slot],
                                  lds[slot]).wait()
            pltpu.make_async_copy(sup_hbm.at[pl.ds(0, ROWS_B)],
                                  rows_v.at[slot], lds[slot]).wait()
            cps = []
            for kk in range(G_S):
                cps.append(pltpu.async_copy(
                    rows_v.at[slot, pl.ds(kk * CH_S, CH_S)],
                    acc_sh.at[sidx_v.at[slot, kk]], sc_sem, add=True))
            for cp in cps:
                cp.wait()

            @pl.when(i + 2 < nblk_s)
            def _refill():
                s_issue(i + 2, slot)

        s_issue(0, 0)
        s_issue(1, 1)

        def s_pair(t, carry):
            s_process(2 * t, 0)
            s_process(2 * t + 1, 1)
            return carry

        lax.fori_loop(0, nblk_s // 2, s_pair, 0)
        if nblk_s % 2:
            s_process(nblk_s - 1, 0)

        plsc.subcore_barrier()

        @pl.when(s == 0)
        def _writeout():
            pltpu.sync_copy(acc_sh, sums_out.at[c])

    return k(sup, st3, z128)


def _tc_pre(sup3, st3, qt3):
    def body(sup_ref, st_ref, qt_ref, psum_ref, cnt_ref, qcnt_ref):
        i = pl.program_id(0)

        @pl.when(i == 0)
        def _init():
            psum_ref[...] = jnp.zeros_like(psum_ref)
            cnt_ref[...] = jnp.zeros_like(cnt_ref)
            qcnt_ref[...] = jnp.zeros_like(qcnt_ref)

        iota = lax.broadcasted_iota(jnp.int32, (N_CLASS, TB2), 0)
        ones8 = jnp.ones((TB2, 8), jnp.float32)
        t8 = st_ref[0]
        ohsum = None
        for r in range(8):
            oh_r = (iota == t8[r:r + 1, :]).astype(jnp.float32)
            ohsum = oh_r if r == 0 else ohsum + oh_r

            @pl.when(i >= CUT_B)
            def _psum(oh_r=oh_r, r=r):
                psum_ref[...] += lax.dot_general(
                    oh_r, sup_ref[0, pl.ds(r * TB2, TB2)],
                    (((1,), (0,)), ((), ())),
                    preferred_element_type=jnp.float32)

        cnt_ref[...] += lax.dot_general(
            ohsum, ones8, (((1,), (0,)), ((), ())),
            preferred_element_type=jnp.float32)

        @pl.when(i < N_QT_BLKS)
        def _qhist():
            q8 = qt_ref[0]
            qsum = None
            for r in range(8):
                oh_r = (iota == q8[r:r + 1, :]).astype(jnp.float32)
                qsum = oh_r if r == 0 else qsum + oh_r
            qcnt_ref[...] += lax.dot_general(
                qsum, ones8, (((1,), (0,)), ((), ())),
                preferred_element_type=jnp.float32)

    return pl.pallas_call(
        body,
        grid=(N_SUP_BLKS,),
        in_specs=[
            pl.BlockSpec((1, RB2, D),
                         lambda i: (jnp.maximum(i, CUT_B), 0, 0)),
            pl.BlockSpec((1, 8, TB2), lambda i: (i, 0, 0)),
            pl.BlockSpec((1, 8, TB2),
                         lambda i: (jnp.minimum(i, N_QT_BLKS - 1), 0, 0)),
        ],
        out_specs=[
            pl.BlockSpec((N_CLASS, D), lambda i: (0, 0)),
            pl.BlockSpec((N_CLASS, 8), lambda i: (0, 0)),
            pl.BlockSpec((N_CLASS, 8), lambda i: (0, 0)),
        ],
        out_shape=[
            jax.ShapeDtypeStruct((N_CLASS, D), jnp.float32),
            jax.ShapeDtypeStruct((N_CLASS, 8), jnp.float32),
            jax.ShapeDtypeStruct((N_CLASS, 8), jnp.float32),
        ],
    )(sup3, st3, qt3)


def _tc_final(q, sums2, psum, cnt, qcnt):
    nq = q.shape[0]
    B = 16000
    grid = nq // B

    def body(q_ref, sums_ref, psum_ref, cnt_ref, qcnt_ref, out_ref):
        s = sums_ref[0] + sums_ref[1] + psum_ref[...]
        cntc = cnt_ref[:, 0:1]
        p = s / cntc
        ones_row = jnp.ones((1, D), jnp.float32)
        pn2 = lax.dot_general(ones_row, p * p, (((1,), (1,)), ((), ())),
                              preferred_element_type=jnp.float32)
        ones8 = jnp.ones((1, 8), jnp.float32)
        pres = lax.dot_general(ones8, qcnt_ref[...], (((1,), (1,)), ((), ())),
                               preferred_element_type=jnp.float32)
        scale = jnp.where(pres > 0, 10.0, 0.0)
        qv = q_ref[...]
        qn2 = jnp.sum(qv * qv, axis=1, keepdims=True)
        dots = lax.dot_general(qv, p, (((1,), (1,)), ((), ())),
                               preferred_element_type=jnp.float32)
        denom = jnp.maximum(jnp.sqrt(qn2 * pn2), 1e-8)
        out_ref[...] = dots / denom * scale

    return pl.pallas_call(
        body,
        grid=(grid,),
        in_specs=[
            pl.BlockSpec((B, D), lambda i: (i, 0)),
            pl.BlockSpec((NC, N_CLASS, D), lambda i: (0, 0, 0)),
            pl.BlockSpec((N_CLASS, D), lambda i: (0, 0)),
            pl.BlockSpec((N_CLASS, 8), lambda i: (0, 0)),
            pl.BlockSpec((N_CLASS, 8), lambda i: (0, 0)),
        ],
        out_specs=pl.BlockSpec((B, N_CLASS), lambda i: (i, 0)),
        out_shape=jax.ShapeDtypeStruct((nq, N_CLASS), jnp.float32),
    )(q, sums2, psum, cnt, qcnt)


def kernel(support_embeddings, support_targets, query_embeddings, query_targets):
    n_sc = CUT_B * RB2
    z128 = jnp.zeros((N_CLASS, D), jnp.float32)
    st3_sc = support_targets[:n_sc].reshape(-1, G_S, CH_S)
    sup3 = support_embeddings.reshape(N_SUP_BLKS, RB2, D)
    st3 = support_targets.reshape(N_SUP_BLKS, 8, TB2)
    qt3 = query_targets.reshape(N_QT_BLKS, 8, TB2)
    sums2 = _sc_stage(support_embeddings, st3_sc, z128)
    psum, cnt, qcnt = _tc_pre(sup3, st3, qt3)
    return _tc_final(query_embeddings, sums2, psum, cnt, qcnt)


# speedup vs baseline: 8.0609x; 1.0011x over previous
import functools

import jax
import jax.numpy as jnp
from jax import lax
from jax.experimental import pallas as pl
from jax.experimental.pallas import tpu as pltpu
from jax.experimental.pallas import tpu_sc as plsc

N_CLASS = 64
D = 128
CH_S = 100
G_S = 4
ROWS_B = G_S * CH_S
NC = 2
NS = 16
NW = NC * NS

TB2 = 2000
RB2 = 8 * TB2
N_SUP_BLKS = 20
CUT_B = 12
N_QT_BLKS = 10


def _sc_stage(sup, st3, z128):
    nblk_s = st3.shape[0] // NW
    mesh = plsc.VectorSubcoreMesh(core_axis_name="c", subcore_axis_name="s")

    @functools.partial(
        pl.kernel,
        mesh=mesh,
        out_type=jax.ShapeDtypeStruct((NC, N_CLASS, D), jnp.float32),
        scratch_types=[
            pltpu.VMEM((2, ROWS_B, D), jnp.float32),
            pltpu.VMEM((2, G_S, CH_S), jnp.int32),
            pltpu.VMEM_SHARED((N_CLASS, D), jnp.float32),
            pltpu.SemaphoreType.DMA,
            pltpu.SemaphoreType.DMA,
            pltpu.SemaphoreType.DMA,
        ],
    )
    def k(sup_hbm, st_hbm, z128_hbm, sums_out,
          rows_v, sidx_v, acc_sh, ld0, ld1, sc_sem):
        c = lax.axis_index("c")
        s = lax.axis_index("s")
        w = c * NS + s
        lds = (ld0, ld1)

        @pl.when(s == 0)
        def _init_shared():
            pltpu.sync_copy(z128_hbm, acc_sh)

        plsc.subcore_barrier()

        def s_issue(i, slot):
            g = i * NW + w
            pltpu.async_copy(st_hbm.at[g], sidx_v.at[slot], lds[slot])
            pltpu.async_copy(sup_hbm.at[pl.ds(g * ROWS_B, ROWS_B)],
                             rows_v.at[slot], lds[slot])

        def s_process(i, slot):
            pltpu.make_async_copy(st_hbm.at[0], sidx_v.at[slot],
                                  lds[slot]).wait()
            pltpu.make_async_copy(sup_hbm.at[pl.ds(0, ROWS_B)],
                                  rows_v.at[slot], lds[slot]).wait()
            cps = []
            for kk in range(G_S):
                cps.append(pltpu.async_copy(
                    rows_v.at[slot, pl.ds(kk * CH_S, CH_S)],
                    acc_sh.at[sidx_v.at[slot, kk]], sc_sem, add=True))
            for cp in cps:
                cp.wait()

            @pl.when(i + 2 < nblk_s)
            def _refill():
                s_issue(i + 2, slot)

        s_issue(0, 0)
        s_issue(1, 1)

        def s_pair(t, carry):
            s_process(2 * t, 0)
            s_process(2 * t + 1, 1)
            return carry

        lax.fori_loop(0, nblk_s // 2, s_pair, 0)
        if nblk_s % 2:
            s_process(nblk_s - 1, 0)

        plsc.subcore_barrier()

        @pl.when(s == 0)
        def _writeout():
            pltpu.sync_copy(acc_sh, sums_out.at[c])

    return k(sup, st3, z128)


def _tc_pre(sup3, st3, qt3):
    def body(sup_ref, st_ref, qt_ref, psum_ref, cnt_ref, qcnt_ref):
        i = pl.program_id(0)

        @pl.when(i == 0)
        def _init():
            psum_ref[...] = jnp.zeros_like(psum_ref)
            cnt_ref[...] = jnp.zeros_like(cnt_ref)
            qcnt_ref[...] = jnp.zeros_like(qcnt_ref)

        iota = lax.broadcasted_iota(jnp.int32, (N_CLASS, TB2), 0)
        ones8 = jnp.ones((TB2, 8), jnp.float32)
        t8 = st_ref[0]
        ohsum = None
        for r in range(8):
            oh_r = (iota == t8[r:r + 1, :]).astype(jnp.float32)
            ohsum = oh_r if r == 0 else ohsum + oh_r

            @pl.when(i >= CUT_B)
            def _psum(oh_r=oh_r, r=r):
                psum_ref[...] += lax.dot_general(
                    oh_r, sup_ref[0, pl.ds(r * TB2, TB2)],
                    (((1,), (0,)), ((), ())),
                    preferred_element_type=jnp.float32)

        cnt_ref[...] += lax.dot_general(
            ohsum, ones8, (((1,), (0,)), ((), ())),
            preferred_element_type=jnp.float32)

        @pl.when(i < N_QT_BLKS)
        def _qhist():
            q8 = qt_ref[0]
            qsum = None
            for r in range(8):
                oh_r = (iota == q8[r:r + 1, :]).astype(jnp.float32)
                qsum = oh_r if r == 0 else qsum + oh_r
            qcnt_ref[...] += lax.dot_general(
                qsum, ones8, (((1,), (0,)), ((), ())),
                preferred_element_type=jnp.float32)

    return pl.pallas_call(
        body,
        grid=(N_SUP_BLKS,),
        in_specs=[
            pl.BlockSpec((1, RB2, D),
                         lambda i: (jnp.maximum(i, CUT_B), 0, 0)),
            pl.BlockSpec((1, 8, TB2), lambda i: (i, 0, 0)),
            pl.BlockSpec((1, 8, TB2),
                         lambda i: (jnp.minimum(i, N_QT_BLKS - 1), 0, 0)),
        ],
        out_specs=[
            pl.BlockSpec((N_CLASS, D), lambda i: (0, 0)),
            pl.BlockSpec((N_CLASS, 8), lambda i: (0, 0)),
            pl.BlockSpec((N_CLASS, 8), lambda i: (0, 0)),
        ],
        out_shape=[
            jax.ShapeDtypeStruct((N_CLASS, D), jnp.float32),
            jax.ShapeDtypeStruct((N_CLASS, 8), jnp.float32),
            jax.ShapeDtypeStruct((N_CLASS, 8), jnp.float32),
        ],
    )(sup3, st3, qt3)


def _tc_final(q, sums2, psum, cnt, qcnt):
    nq = q.shape[0]
    B = 20000
    grid = nq // B

    def body(q_ref, sums_ref, psum_ref, cnt_ref, qcnt_ref, out_ref):
        s = sums_ref[0] + sums_ref[1] + psum_ref[...]
        cntc = cnt_ref[:, 0:1]
        p = s / cntc
        ones_row = jnp.ones((1, D), jnp.float32)
        pn2 = lax.dot_general(ones_row, p * p, (((1,), (1,)), ((), ())),
                              preferred_element_type=jnp.float32)
        ones8 = jnp.ones((1, 8), jnp.float32)
        pres = lax.dot_general(ones8, qcnt_ref[...], (((1,), (1,)), ((), ())),
                               preferred_element_type=jnp.float32)
        scale = jnp.where(pres > 0, 10.0, 0.0)
        qv = q_ref[...]
        qn2 = jnp.sum(qv * qv, axis=1, keepdims=True)
        dots = lax.dot_general(qv, p, (((1,), (1,)), ((), ())),
                               preferred_element_type=jnp.float32)
        denom = jnp.maximum(jnp.sqrt(qn2 * pn2), 1e-8)
        out_ref[...] = dots / denom * scale

    return pl.pallas_call(
        body,
        grid=(grid,),
        in_specs=[
            pl.BlockSpec((B, D), lambda i: (i, 0)),
            pl.BlockSpec((NC, N_CLASS, D), lambda i: (0, 0, 0)),
            pl.BlockSpec((N_CLASS, D), lambda i: (0, 0)),
            pl.BlockSpec((N_CLASS, 8), lambda i: (0, 0)),
            pl.BlockSpec((N_CLASS, 8), lambda i: (0, 0)),
        ],
        out_specs=pl.BlockSpec((B, N_CLASS), lambda i: (i, 0)),
        out_shape=jax.ShapeDtypeStruct((nq, N_CLASS), jnp.float32),
    )(q, sums2, psum, cnt, qcnt)


def kernel(support_embeddings, support_targets, query_embeddings, query_targets):
    n_sc = CUT_B * RB2
    z128 = jnp.zeros((N_CLASS, D), jnp.float32)
    st3_sc = support_targets[:n_sc].reshape(-1, G_S, CH_S)
    sup3 = support_embeddings.reshape(N_SUP_BLKS, RB2, D)
    st3 = support_targets.reshape(N_SUP_BLKS, 8, TB2)
    qt3 = query_targets.reshape(N_QT_BLKS, 8, TB2)
    sums2 = _sc_stage(support_embeddings, st3_sc, z128)
    psum, cnt, qcnt = _tc_pre(sup3, st3, qt3)
    return _tc_final(query_embeddings, sums2, psum, cnt, qcnt)
